# Initial kernel scaffold; baseline (speedup 1.0000x reference)
import jax
import jax.numpy as jnp
from jax.experimental import pallas as pl


def kernel(x, edge_index, edge_attr, be1_W, be1_b, be2_W, be2_b, m1_W, m1_b, m2_W, m2_b, eps, bn_g, bn_b, fc1_W, fc1_b, bnl_g, bnl_b, fc2_W, fc2_b):
    raise NotImplementedError("write your pallas kernel here")


# speedup vs baseline: 1.5036x; 1.5036x over previous
import functools

import jax
import jax.numpy as jnp
from jax import lax
from jax.experimental import pallas as pl
from jax.experimental.pallas import tpu as pltpu
from jax.experimental.pallas import tpu_sc as plsc

N_ = 10000
E_ = 160000
D = 256
H = 128
DE_ = 16
L_ = 4

BE = 640
EB = E_ // BE
BN_ = 400
NB = N_ // BN_

NPAD = 10240
NPT = NPAD // 16
EPT = E_ // 16
CH = 80
NCH = EPT // CH


def _edge_body(ea_ref, w1_ref, b1_ref, w2_ref, b2_ref, out_ref):
    hid = jnp.maximum(
        jnp.dot(ea_ref[...], w1_ref[...], preferred_element_type=jnp.float32)
        + b1_ref[...], 0.0)
    out_ref[...] = (
        jnp.dot(hid, w2_ref[...], preferred_element_type=jnp.float32)
        + b2_ref[...])


def _edge_mlp(ea, w1, b1, w2, b2):
    return pl.pallas_call(
        _edge_body,
        grid=(EB,),
        in_specs=[
            pl.BlockSpec((BE, DE_), lambda i: (i, 0)),
            pl.BlockSpec((DE_, D), lambda i: (0, 0)),
            pl.BlockSpec((1, D), lambda i: (0, 0)),
            pl.BlockSpec((D, D), lambda i: (0, 0)),
            pl.BlockSpec((1, D), lambda i: (0, 0)),
        ],
        out_specs=pl.BlockSpec((BE, D), lambda i: (i, 0)),
        out_shape=jax.ShapeDtypeStruct((E_, D), jnp.float32),
    )(ea, w1, b1.reshape(1, D), w2, b2.reshape(1, D))


def _node_body(ha_ref, hb_ref, aa_ref, ab_ref, w1_ref, b1_ref, w2_ref, b2_ref,
               ep_ref, v_ref, s1_ref, s2_ref):
    i = pl.program_id(0)
    hblk = jnp.concatenate([ha_ref[...], hb_ref[...]], axis=1)
    ablk = jnp.concatenate([aa_ref[0], ab_ref[0]], axis=1)
    u = ep_ref[...] * hblk + ablk
    t = jnp.maximum(
        jnp.dot(u, w1_ref[...], preferred_element_type=jnp.float32)
        + b1_ref[...], 0.0)
    v = (jnp.dot(t, w2_ref[...], preferred_element_type=jnp.float32)
         + b2_ref[...])
    v_ref[...] = v
    p1 = jnp.sum(v, axis=0, keepdims=True)
    p2 = jnp.sum(v * v, axis=0, keepdims=True)

    @pl.when(i == 0)
    def _():
        s1_ref[...] = p1
        s2_ref[...] = p2

    @pl.when(i > 0)
    def _():
        s1_ref[...] += p1
        s2_ref[...] += p2


def _node_mlp(h2, aggr2, w1, b1, w2, b2, epsv):
    return pl.pallas_call(
        _node_body,
        grid=(NB,),
        in_specs=[
            pl.BlockSpec((BN_, H), lambda i: (i, 0)),
            pl.BlockSpec((BN_, H), lambda i: (NB + i, 0)),
            pl.BlockSpec((1, BN_, H), lambda i: (0, i, 0)),
            pl.BlockSpec((1, BN_, H), lambda i: (1, i, 0)),
            pl.BlockSpec((D, D), lambda i: (0, 0)),
            pl.BlockSpec((1, D), lambda i: (0, 0)),
            pl.BlockSpec((D, D), lambda i: (0, 0)),
            pl.BlockSpec((1, D), lambda i: (0, 0)),
            pl.BlockSpec((1, 1), lambda i: (0, 0)),
        ],
        out_specs=[
            pl.BlockSpec((BN_, D), lambda i: (i, 0)),
            pl.BlockSpec((1, D), lambda i: (0, 0)),
            pl.BlockSpec((1, D), lambda i: (0, 0)),
        ],
        out_shape=[
            jax.ShapeDtypeStruct((N_, D), jnp.float32),
            jax.ShapeDtypeStruct((1, D), jnp.float32),
            jax.ShapeDtypeStruct((1, D), jnp.float32),
        ],
    )(h2, h2, aggr2, aggr2, w1, b1.reshape(1, D), w2, b2.reshape(1, D),
      epsv.reshape(1, 1))


def _bnap_body(v_ref, g_ref, mu_ref, var_ref, b_ref, out_ref):
    out_ref[...] = jnp.maximum(
        g_ref[...] * (v_ref[...] - mu_ref[...])
        / jnp.sqrt(var_ref[...] + 1e-5) + b_ref[...], 0.0)


def _bn_apply(v, g, mu, var, b):
    return pl.pallas_call(
        _bnap_body,
        grid=(2, NB),
        in_specs=[
            pl.BlockSpec((BN_, H), lambda c, i: (i, c)),
            pl.BlockSpec((1, H), lambda c, i: (0, c)),
            pl.BlockSpec((1, H), lambda c, i: (0, c)),
            pl.BlockSpec((1, H), lambda c, i: (0, c)),
            pl.BlockSpec((1, H), lambda c, i: (0, c)),
        ],
        out_specs=pl.BlockSpec((BN_, H), lambda c, i: (c * NB + i, 0)),
        out_shape=jax.ShapeDtypeStruct((2 * N_, H), jnp.float32),
    )(v, g.reshape(1, D), mu.reshape(1, D), var.reshape(1, D),
      b.reshape(1, D))


def _heada_body(ha_ref, hb_ref, w_ref, b_ref, z_ref, s1_ref, s2_ref):
    i = pl.program_id(0)
    hblk = jnp.concatenate([ha_ref[...], hb_ref[...]], axis=1)
    z = (jnp.dot(hblk, w_ref[...], preferred_element_type=jnp.float32)
         + b_ref[...])
    z_ref[...] = z
    p1 = jnp.sum(z, axis=0, keepdims=True)
    p2 = jnp.sum(z * z, axis=0, keepdims=True)

    @pl.when(i == 0)
    def _():
        s1_ref[...] = p1
        s2_ref[...] = p2

    @pl.when(i > 0)
    def _():
        s1_ref[...] += p1
        s2_ref[...] += p2


def _head_a(h2, w, b):
    return pl.pallas_call(
        _heada_body,
        grid=(NB,),
        in_specs=[
            pl.BlockSpec((BN_, H), lambda i: (i, 0)),
            pl.BlockSpec((BN_, H), lambda i: (NB + i, 0)),
            pl.BlockSpec((D, D), lambda i: (0, 0)),
            pl.BlockSpec((1, D), lambda i: (0, 0)),
        ],
        out_specs=[
            pl.BlockSpec((BN_, D), lambda i: (i, 0)),
            pl.BlockSpec((1, D), lambda i: (0, 0)),
            pl.BlockSpec((1, D), lambda i: (0, 0)),
        ],
        out_shape=[
            jax.ShapeDtypeStruct((N_, D), jnp.float32),
            jax.ShapeDtypeStruct((1, D), jnp.float32),
            jax.ShapeDtypeStruct((1, D), jnp.float32),
        ],
    )(h2, h2, w, b.reshape(1, D))


def _headb_body(z_ref, w_ref, b_ref, out_ref):
    out_ref[...] = (
        jnp.dot(z_ref[...], w_ref[...], preferred_element_type=jnp.float32)
        + b_ref[...])


def _head_b(zz, w, b, nc):
    return pl.pallas_call(
        _headb_body,
        grid=(NB,),
        in_specs=[
            pl.BlockSpec((BN_, D), lambda i: (i, 0)),
            pl.BlockSpec((D, nc), lambda i: (0, 0)),
            pl.BlockSpec((1, nc), lambda i: (0, 0)),
        ],
        out_specs=pl.BlockSpec((BN_, nc), lambda i: (i, 0)),
        out_shape=jax.ShapeDtypeStruct((N_, nc), jnp.float32),
    )(zz, w, b.reshape(1, nc))


_sc_mesh = plsc.VectorSubcoreMesh(core_axis_name="c", subcore_axis_name="s")


@functools.partial(
    pl.kernel,
    mesh=_sc_mesh,
    out_type=jax.ShapeDtypeStruct((2, NPAD, H), jnp.float32),
    scratch_types=[
        pltpu.VMEM((CH,), jnp.int32),
        pltpu.VMEM((CH,), jnp.int32),
        pltpu.VMEM((CH,), jnp.int32),
        pltpu.VMEM((CH, H), jnp.float32),
        pltpu.VMEM((CH, H), jnp.float32),
        pltpu.VMEM_SHARED((NPAD, H), jnp.float32),
        pltpu.SemaphoreType.DMA,
        pltpu.SemaphoreType.DMA,
    ],
)
def _sc_message(h2, e2, gsrc, eix, dst, out,
                gix_v, eix_v, dix_v, rows_v, e_v, aggr_sh,
                sem1, sem2):
    c = lax.axis_index("c")
    s = lax.axis_index("s")

    def _z(i, carry):
        for j in range(H // 16):
            rows_v[i, pl.ds(j * 16, 16)] = jnp.zeros((16,), jnp.float32)
        return carry
    lax.fori_loop(0, CH, _z, 0)

    def _zc(t, carry):
        pltpu.sync_copy(rows_v, aggr_sh.at[pl.ds(s * NPT + t * CH, CH)])
        return carry
    lax.fori_loop(0, NPT // CH, _zc, 0)
    plsc.subcore_barrier()

    def _chunk(k, carry):
        base = s * EPT + k * CH
        pltpu.sync_copy(gsrc.at[pl.ds(c * E_ + base, CH)], gix_v)
        pltpu.sync_copy(dst.at[pl.ds(base, CH)], dix_v)
        pltpu.sync_copy(eix.at[pl.ds(c * E_ + base, CH)], eix_v)
        cp1 = pltpu.async_copy(h2.at[gix_v], rows_v, sem1)
        cp2 = pltpu.async_copy(e2.at[eix_v], e_v, sem2)
        cp1.wait()
        cp2.wait()

        def _row(i, c2):
            for j in range(H // 16):
                sl = pl.ds(j * 16, 16)
                rows_v[i, sl] = jnp.maximum(rows_v[i, sl] + e_v[i, sl], 0.0)
            return c2
        lax.fori_loop(0, CH, _row, 0)
        pltpu.sync_copy(rows_v, aggr_sh.at[dix_v], add=True)
        return carry
    lax.fori_loop(0, NCH, _chunk, 0)
    plsc.subcore_barrier()

    def _wb(t, carry):
        r0 = s * NPT + t * CH
        pltpu.sync_copy(aggr_sh.at[pl.ds(r0, CH)], e_v)
        pltpu.sync_copy(e_v, out.at[c, pl.ds(r0, CH)])
        return carry
    lax.fori_loop(0, NPT // CH, _wb, 0)


def kernel(x, edge_index, edge_attr, be1_W, be1_b, be2_W, be2_b, m1_W, m1_b,
           m2_W, m2_b, eps, bn_g, bn_b, fc1_W, fc1_b, bnl_g, bnl_b,
           fc2_W, fc2_b):
    nc = fc2_W.shape[1]
    src = edge_index[0].astype(jnp.int32)
    dst = edge_index[1].astype(jnp.int32)
    perm = jnp.argsort(dst, stable=True).astype(jnp.int32)
    sdst = dst[perm]
    psrc = src[perm]
    gsrc = jnp.concatenate([psrc, psrc + N_])
    eix = jnp.concatenate([2 * perm, 2 * perm + 1])
    h2 = jnp.concatenate([x[:, :H], x[:, H:]], axis=0)

    es = [_edge_mlp(edge_attr, be1_W[i], be1_b[i], be2_W[i], be2_b[i])
          for i in range(L_)]

    for i in range(L_):
        e2 = es[i].reshape(2 * E_, H)
        aggr2 = _sc_message(h2, e2, gsrc, eix, sdst)
        v, s1, s2 = _node_mlp(h2, aggr2, m1_W[i], m1_b[i], m2_W[i], m2_b[i],
                              1.0 + eps[i])
        v64 = v.astype(jnp.float64)
        mu = jnp.mean(v64, axis=0).astype(jnp.float32)
        var = jnp.var(v64, axis=0).astype(jnp.float32)
        h2 = _bn_apply(v, bn_g[i], mu, var, bn_b[i])

    z, s1, s2 = _head_a(h2, fc1_W, fc1_b)
    z64 = z.astype(jnp.float64)
    mu = jnp.mean(z64, axis=0).astype(jnp.float32)
    var = jnp.var(z64, axis=0).astype(jnp.float32)
    zz = jax.nn.relu(bnl_g * (z - mu) / jnp.sqrt(var + 1e-5) + bnl_b)
    return _head_b(zz, fc2_W, fc2_b, nc)


# speedup vs baseline: 1.5044x; 1.0006x over previous
import functools

import jax
import jax.numpy as jnp
from jax import lax
from jax.experimental import pallas as pl
from jax.experimental.pallas import tpu as pltpu
from jax.experimental.pallas import tpu_sc as plsc

N_ = 10000
E_ = 160000
D = 256
H = 128
DE_ = 16
L_ = 4

BE = 640
EB = E_ // BE
BN_ = 400
NB = N_ // BN_

NPAD = 10240
NPT = NPAD // 16
EPT = E_ // 16
CH = 80
NCH = EPT // CH


def _edge_body(ea_ref, w1_ref, b1_ref, w2_ref, b2_ref, out_ref):
    hid = jnp.maximum(
        jnp.dot(ea_ref[...], w1_ref[...], preferred_element_type=jnp.float32)
        + b1_ref[...], 0.0)
    out_ref[...] = (
        jnp.dot(hid, w2_ref[...], preferred_element_type=jnp.float32)
        + b2_ref[...])


def _edge_mlp(ea, w1, b1, w2, b2):
    return pl.pallas_call(
        _edge_body,
        grid=(EB,),
        in_specs=[
            pl.BlockSpec((BE, DE_), lambda i: (i, 0)),
            pl.BlockSpec((DE_, D), lambda i: (0, 0)),
            pl.BlockSpec((1, D), lambda i: (0, 0)),
            pl.BlockSpec((D, D), lambda i: (0, 0)),
            pl.BlockSpec((1, D), lambda i: (0, 0)),
        ],
        out_specs=pl.BlockSpec((BE, D), lambda i: (i, 0)),
        out_shape=jax.ShapeDtypeStruct((E_, D), jnp.float32),
    )(ea, w1, b1.reshape(1, D), w2, b2.reshape(1, D))


def _node_body(ha_ref, hb_ref, aa_ref, ab_ref, w1_ref, b1_ref, w2_ref, b2_ref,
               ep_ref, v_ref, s1_ref, s2_ref):
    i = pl.program_id(0)
    hblk = jnp.concatenate([ha_ref[...], hb_ref[...]], axis=1)
    ablk = jnp.concatenate([aa_ref[0], ab_ref[0]], axis=1)
    u = ep_ref[...] * hblk + ablk
    t = jnp.maximum(
        jnp.dot(u, w1_ref[...], preferred_element_type=jnp.float32)
        + b1_ref[...], 0.0)
    v = (jnp.dot(t, w2_ref[...], preferred_element_type=jnp.float32)
         + b2_ref[...])
    v_ref[...] = v
    p1 = jnp.sum(v, axis=0, keepdims=True)
    p2 = jnp.sum(v * v, axis=0, keepdims=True)

    @pl.when(i == 0)
    def _():
        s1_ref[...] = p1
        s2_ref[...] = p2

    @pl.when(i > 0)
    def _():
        s1_ref[...] += p1
        s2_ref[...] += p2


def _node_mlp(h2, aggr2, w1, b1, w2, b2, epsv):
    return pl.pallas_call(
        _node_body,
        grid=(NB,),
        in_specs=[
            pl.BlockSpec((BN_, H), lambda i: (i, 0)),
            pl.BlockSpec((BN_, H), lambda i: (NB + i, 0)),
            pl.BlockSpec((1, BN_, H), lambda i: (0, i, 0)),
            pl.BlockSpec((1, BN_, H), lambda i: (1, i, 0)),
            pl.BlockSpec((D, D), lambda i: (0, 0)),
            pl.BlockSpec((1, D), lambda i: (0, 0)),
            pl.BlockSpec((D, D), lambda i: (0, 0)),
            pl.BlockSpec((1, D), lambda i: (0, 0)),
            pl.BlockSpec((1, 1), lambda i: (0, 0)),
        ],
        out_specs=[
            pl.BlockSpec((BN_, D), lambda i: (i, 0)),
            pl.BlockSpec((1, D), lambda i: (0, 0)),
            pl.BlockSpec((1, D), lambda i: (0, 0)),
        ],
        out_shape=[
            jax.ShapeDtypeStruct((N_, D), jnp.float32),
            jax.ShapeDtypeStruct((1, D), jnp.float32),
            jax.ShapeDtypeStruct((1, D), jnp.float32),
        ],
    )(h2, h2, aggr2, aggr2, w1, b1.reshape(1, D), w2, b2.reshape(1, D),
      epsv.reshape(1, 1))


def _bnap_body(v_ref, g_ref, mu_ref, var_ref, b_ref, out_ref):
    out_ref[...] = jnp.maximum(
        g_ref[...] * (v_ref[...] - mu_ref[...])
        / jnp.sqrt(var_ref[...] + 1e-5) + b_ref[...], 0.0)


def _bn_apply(v, g, mu, var, b):
    return pl.pallas_call(
        _bnap_body,
        grid=(2, NB),
        in_specs=[
            pl.BlockSpec((BN_, H), lambda c, i: (i, c)),
            pl.BlockSpec((1, H), lambda c, i: (0, c)),
            pl.BlockSpec((1, H), lambda c, i: (0, c)),
            pl.BlockSpec((1, H), lambda c, i: (0, c)),
            pl.BlockSpec((1, H), lambda c, i: (0, c)),
        ],
        out_specs=pl.BlockSpec((BN_, H), lambda c, i: (c * NB + i, 0)),
        out_shape=jax.ShapeDtypeStruct((2 * N_, H), jnp.float32),
    )(v, g.reshape(1, D), mu.reshape(1, D), var.reshape(1, D),
      b.reshape(1, D))


def _heada_body(ha_ref, hb_ref, w_ref, b_ref, z_ref, s1_ref, s2_ref):
    i = pl.program_id(0)
    hblk = jnp.concatenate([ha_ref[...], hb_ref[...]], axis=1)
    z = (jnp.dot(hblk, w_ref[...], preferred_element_type=jnp.float32)
         + b_ref[...])
    z_ref[...] = z
    p1 = jnp.sum(z, axis=0, keepdims=True)
    p2 = jnp.sum(z * z, axis=0, keepdims=True)

    @pl.when(i == 0)
    def _():
        s1_ref[...] = p1
        s2_ref[...] = p2

    @pl.when(i > 0)
    def _():
        s1_ref[...] += p1
        s2_ref[...] += p2


def _head_a(h2, w, b):
    return pl.pallas_call(
        _heada_body,
        grid=(NB,),
        in_specs=[
            pl.BlockSpec((BN_, H), lambda i: (i, 0)),
            pl.BlockSpec((BN_, H), lambda i: (NB + i, 0)),
            pl.BlockSpec((D, D), lambda i: (0, 0)),
            pl.BlockSpec((1, D), lambda i: (0, 0)),
        ],
        out_specs=[
            pl.BlockSpec((BN_, D), lambda i: (i, 0)),
            pl.BlockSpec((1, D), lambda i: (0, 0)),
            pl.BlockSpec((1, D), lambda i: (0, 0)),
        ],
        out_shape=[
            jax.ShapeDtypeStruct((N_, D), jnp.float32),
            jax.ShapeDtypeStruct((1, D), jnp.float32),
            jax.ShapeDtypeStruct((1, D), jnp.float32),
        ],
    )(h2, h2, w, b.reshape(1, D))


def _headb_body(z_ref, w_ref, b_ref, out_ref):
    out_ref[...] = (
        jnp.dot(z_ref[...], w_ref[...], preferred_element_type=jnp.float32)
        + b_ref[...])


def _head_b(zz, w, b, nc):
    return pl.pallas_call(
        _headb_body,
        grid=(NB,),
        in_specs=[
            pl.BlockSpec((BN_, D), lambda i: (i, 0)),
            pl.BlockSpec((D, nc), lambda i: (0, 0)),
            pl.BlockSpec((1, nc), lambda i: (0, 0)),
        ],
        out_specs=pl.BlockSpec((BN_, nc), lambda i: (i, 0)),
        out_shape=jax.ShapeDtypeStruct((N_, nc), jnp.float32),
    )(zz, w, b.reshape(1, nc))


_sc_mesh = plsc.VectorSubcoreMesh(core_axis_name="c", subcore_axis_name="s")


@functools.partial(
    pl.kernel,
    mesh=_sc_mesh,
    out_type=jax.ShapeDtypeStruct((2, NPAD, H), jnp.float32),
    scratch_types=[
        pltpu.VMEM((CH,), jnp.int32),
        pltpu.VMEM((CH,), jnp.int32),
        pltpu.VMEM((CH,), jnp.int32),
        pltpu.VMEM((CH, H), jnp.float32),
        pltpu.VMEM((CH, H), jnp.float32),
        pltpu.VMEM_SHARED((NPAD, H), jnp.float32),
        pltpu.SemaphoreType.DMA,
        pltpu.SemaphoreType.DMA,
    ],
)
def _sc_message(h2, e2, gsrc, eix, dst, out,
                gix_v, eix_v, dix_v, rows_v, e_v, aggr_sh,
                sem1, sem2):
    c = lax.axis_index("c")
    s = lax.axis_index("s")

    def _z(i, carry):
        for j in range(H // 16):
            rows_v[i, pl.ds(j * 16, 16)] = jnp.zeros((16,), jnp.float32)
        return carry
    lax.fori_loop(0, CH, _z, 0)

    def _zc(t, carry):
        pltpu.sync_copy(rows_v, aggr_sh.at[pl.ds(s * NPT + t * CH, CH)])
        return carry
    lax.fori_loop(0, NPT // CH, _zc, 0)
    plsc.subcore_barrier()

    def _chunk(k, carry):
        base = s * EPT + k * CH
        pltpu.sync_copy(gsrc.at[pl.ds(c * E_ + base, CH)], gix_v)
        pltpu.sync_copy(dst.at[pl.ds(base, CH)], dix_v)
        pltpu.sync_copy(eix.at[pl.ds(c * E_ + base, CH)], eix_v)
        cp1 = pltpu.async_copy(h2.at[gix_v], rows_v, sem1)
        cp2 = pltpu.async_copy(e2.at[eix_v], e_v, sem2)
        cp1.wait()
        cp2.wait()

        def _row(i, c2):
            for j in range(H // 16):
                sl = pl.ds(j * 16, 16)
                rows_v[i, sl] = jnp.maximum(rows_v[i, sl] + e_v[i, sl], 0.0)
            return c2
        lax.fori_loop(0, CH, _row, 0)
        pltpu.sync_copy(rows_v, aggr_sh.at[dix_v], add=True)
        return carry
    lax.fori_loop(0, NCH, _chunk, 0)
    plsc.subcore_barrier()

    def _wb(t, carry):
        r0 = s * NPT + t * CH
        pltpu.sync_copy(aggr_sh.at[pl.ds(r0, CH)], e_v)
        pltpu.sync_copy(e_v, out.at[c, pl.ds(r0, CH)])
        return carry
    lax.fori_loop(0, NPT // CH, _wb, 0)


def kernel(x, edge_index, edge_attr, be1_W, be1_b, be2_W, be2_b, m1_W, m1_b,
           m2_W, m2_b, eps, bn_g, bn_b, fc1_W, fc1_b, bnl_g, bnl_b,
           fc2_W, fc2_b):
    nc = fc2_W.shape[1]
    src = edge_index[0].astype(jnp.int32)
    dst = edge_index[1].astype(jnp.int32)
    perm = jnp.argsort(dst, stable=True).astype(jnp.int32)
    sdst = dst[perm]
    psrc = src[perm]
    gsrc = jnp.concatenate([psrc, psrc + N_])
    eix = jnp.concatenate([2 * perm, 2 * perm + 1])
    h2 = jnp.concatenate([x[:, :H], x[:, H:]], axis=0)

    es = [_edge_mlp(edge_attr, be1_W[i], be1_b[i], be2_W[i], be2_b[i])
          for i in range(L_)]

    for i in range(L_):
        e2 = es[i].reshape(2 * E_, H)
        aggr2 = _sc_message(h2, e2, gsrc, eix, sdst)
        v, s1, s2 = _node_mlp(h2, aggr2, m1_W[i], m1_b[i], m2_W[i], m2_b[i],
                              1.0 + eps[i])
        mu = jnp.mean(v, axis=0)
        var = jnp.var(v, axis=0)
        h2 = _bn_apply(v, bn_g[i], mu, var, bn_b[i])

    z, s1, s2 = _head_a(h2, fc1_W, fc1_b)
    mu = jnp.mean(z, axis=0)
    var = jnp.var(z, axis=0)
    zz = jax.nn.relu(bnl_g * (z - mu) / jnp.sqrt(var + 1e-5) + bnl_b)
    return _head_b(zz, fc2_W, fc2_b, nc)
